# baseline (speedup 1.0000x reference)
import jax
import jax.numpy as jnp
from jax.experimental import pallas as pl
from jax.experimental.pallas import tpu as pltpu

D = 512
B = 64
T = 3
N = 50000
BLK = 2000
NB = N // BLK
NEG = -1e30
W = 8


def _split(a):
    hi = a.astype(jnp.bfloat16)
    lo = (a - hi.astype(jnp.float32)).astype(jnp.bfloat16)
    return hi, lo


def _body(win_ref, batch_ref, x_ref, wq_ref, wr_ref, b_ref,
          out_ref,
          h_ref, c_ref, r_ref, m_ref, s_ref, u_ref):
    t = pl.program_id(0)
    i = pl.program_id(1)

    @pl.when(i == 0)
    def _lstm_and_init():
        first = (t == 0)
        h_prev = jnp.where(first, 0.0, h_ref[...])
        c_prev = jnp.where(first, 0.0, c_ref[...])
        r_prev = jnp.where(first, 0.0, r_ref[...])

        def d(u, v):
            return jax.lax.dot_general(u, v, (((1,), (0,)), ((), ())),
                                       preferred_element_type=jnp.float32)

        hh_, hl_ = _split(h_prev)
        rh_, rl_ = _split(r_prev)
        qh, ql = _split(wq_ref[...])
        sh, sl = _split(wr_ref[...])
        gates = (d(hh_, qh) + d(hh_, ql) + d(hl_, qh)
                 + d(rh_, sh) + d(rh_, sl) + d(rl_, sh)
                 + b_ref[...])
        gi = jax.nn.sigmoid(gates[:, 0 * D:1 * D])
        gf = jax.nn.sigmoid(gates[:, 1 * D:2 * D])
        gg = jnp.tanh(gates[:, 2 * D:3 * D])
        go = jax.nn.sigmoid(gates[:, 3 * D:4 * D])
        c_new = gf * c_prev + gi * gg
        h_new = go * jnp.tanh(c_new)
        h_ref[...] = h_new
        c_ref[...] = c_new
        m_ref[...] = jnp.full((B, 1), NEG, jnp.float32)
        s_ref[...] = jnp.zeros((B, 1), jnp.float32)
        u_ref[...] = jnp.zeros((B, D), jnp.float32)

    x_blk = x_ref[...]
    xh, xl = _split(x_blk)
    seg = batch_ref[0]
    w0 = win_ref[i, 0]
    nw = win_ref[i, 1]

    def win_body(k, carry):
        w = (w0 + k) * W
        h8 = h_ref[pl.ds(w, W), :]
        h8h, h8l = _split(h8)

        def dxt(u, v):
            return jax.lax.dot_general(u, v, (((1,), (1,)), ((), ())),
                                       preferred_element_type=jnp.float32)

        e8 = dxt(h8h, xh) + dxt(h8h, xl) + dxt(h8l, xh)
        mask = (seg - w) == jax.lax.broadcasted_iota(jnp.int32, (W, BLK), 0)
        e_m = jnp.where(mask, e8, NEG)
        m_old = m_ref[pl.ds(w, W), :]
        m_new = jnp.maximum(m_old, jnp.max(e_m, axis=1, keepdims=True))
        p8 = jnp.where(mask, jnp.exp(e8 - m_new), 0.0)
        scale = jnp.exp(m_old - m_new)
        s_ref[pl.ds(w, W), :] = (s_ref[pl.ds(w, W), :] * scale
                                 + jnp.sum(p8, axis=1, keepdims=True))
        ph, plo = _split(p8)

        def dp(u, v):
            return jax.lax.dot_general(u, v, (((1,), (0,)), ((), ())),
                                       preferred_element_type=jnp.float32)

        u_ref[pl.ds(w, W), :] = (u_ref[pl.ds(w, W), :] * scale
                                 + dp(ph, xh) + dp(ph, xl) + dp(plo, xh))
        m_ref[pl.ds(w, W), :] = m_new
        return carry

    jax.lax.fori_loop(0, nw, win_body, 0)

    @pl.when(i == NB - 1)
    def _finalize():
        r = u_ref[...] / (s_ref[...] + 1e-16)
        r_ref[...] = r

        @pl.when(t == T - 1)
        def _write_out():
            out_ref[:, :D] = h_ref[...]
            out_ref[:, D:] = r


def kernel(x, batch, W_ih, W_hh, b_ih, b_hh):
    batch = batch.astype(jnp.int32)
    batch3 = batch.reshape(NB, 1, BLK)
    idx = jnp.arange(NB)
    lo = batch[idx * BLK]
    hi = batch[idx * BLK + (BLK - 1)]
    w0 = lo // W
    nw = hi // W - lo // W + 1
    win = jnp.stack([w0, nw], axis=1).astype(jnp.int32)
    wq = W_ih.T[:D] + W_hh.T
    wr = W_ih.T[D:]
    bias = (b_ih + b_hh).reshape(1, 4 * D)
    grid_spec = pltpu.PrefetchScalarGridSpec(
        num_scalar_prefetch=1,
        grid=(T, NB),
        in_specs=[
            pl.BlockSpec((1, 1, BLK), lambda t, i, w: (i, 0, 0)),
            pl.BlockSpec((BLK, D), lambda t, i, w: (i, 0)),
            pl.BlockSpec((D, 4 * D), lambda t, i, w: (0, 0)),
            pl.BlockSpec((D, 4 * D), lambda t, i, w: (0, 0)),
            pl.BlockSpec((1, 4 * D), lambda t, i, w: (0, 0)),
        ],
        out_specs=pl.BlockSpec((B, 2 * D), lambda t, i, w: (0, 0)),
        scratch_shapes=[
            pltpu.VMEM((B, D), jnp.float32),
            pltpu.VMEM((B, D), jnp.float32),
            pltpu.VMEM((B, D), jnp.float32),
            pltpu.VMEM((B, 1), jnp.float32),
            pltpu.VMEM((B, 1), jnp.float32),
            pltpu.VMEM((B, D), jnp.float32),
        ],
    )
    return pl.pallas_call(
        _body,
        grid_spec=grid_spec,
        out_shape=jax.ShapeDtypeStruct((B, 2 * D), jnp.float32),
        compiler_params=pltpu.CompilerParams(
            dimension_semantics=("arbitrary", "arbitrary"),
        ),
    )(win, batch3, x, wq, wr, bias)


# speedup vs baseline: 1.1820x; 1.1820x over previous
import jax
import jax.numpy as jnp
from jax.experimental import pallas as pl
from jax.experimental.pallas import tpu as pltpu

D = 512
B = 64
T = 3
N = 50000
BLK = 2000
NB = N // BLK
NEG = -1e30
W = 16


def _split(a):
    hi = a.astype(jnp.bfloat16)
    lo = (a - hi.astype(jnp.float32)).astype(jnp.bfloat16)
    return hi, lo


def _body(win_ref, batch_ref, x_ref, wq_ref, wr_ref, b_ref,
          out_ref,
          h_ref, c_ref, r_ref, m_ref, s_ref, u_ref):
    t = pl.program_id(0)
    i = pl.program_id(1)

    @pl.when(i == 0)
    def _lstm_and_init():
        first = (t == 0)
        h_prev = jnp.where(first, 0.0, h_ref[...])
        c_prev = jnp.where(first, 0.0, c_ref[...])
        r_prev = jnp.where(first, 0.0, r_ref[...])

        def d(u, v):
            return jax.lax.dot_general(u, v, (((1,), (0,)), ((), ())),
                                       preferred_element_type=jnp.float32)

        hh_, hl_ = _split(h_prev)
        rh_, rl_ = _split(r_prev)
        qh, ql = _split(wq_ref[...])
        sh, sl = _split(wr_ref[...])
        gates = (d(hh_, qh) + d(hh_, ql) + d(hl_, qh)
                 + d(rh_, sh) + d(rh_, sl) + d(rl_, sh)
                 + b_ref[...])
        gi = jax.nn.sigmoid(gates[:, 0 * D:1 * D])
        gf = jax.nn.sigmoid(gates[:, 1 * D:2 * D])
        gg = jnp.tanh(gates[:, 2 * D:3 * D])
        go = jax.nn.sigmoid(gates[:, 3 * D:4 * D])
        c_new = gf * c_prev + gi * gg
        h_new = go * jnp.tanh(c_new)
        h_ref[...] = h_new
        c_ref[...] = c_new
        m_ref[...] = jnp.full((B, 1), NEG, jnp.float32)
        s_ref[...] = jnp.zeros((B, 1), jnp.float32)
        u_ref[...] = jnp.zeros((B, D), jnp.float32)

    x_blk = x_ref[...]
    xh, xl = _split(x_blk)
    seg = batch_ref[0]
    w8 = win_ref[i, 0]
    fast = win_ref[i, 1] == 1

    def update(w, wsz):
        hw = h_ref[pl.ds(w, wsz), :]
        hwh, hwl = _split(hw)

        def dxt(u, v):
            return jax.lax.dot_general(u, v, (((1,), (1,)), ((), ())),
                                       preferred_element_type=jnp.float32)

        e = dxt(hwh, xh) + dxt(hwh, xl) + dxt(hwl, xh)
        mask = (seg - w) == jax.lax.broadcasted_iota(jnp.int32, (wsz, BLK), 0)
        e_m = jnp.where(mask, e, NEG)
        m_old = m_ref[pl.ds(w, wsz), :]
        m_new = jnp.maximum(m_old, jnp.max(e_m, axis=1, keepdims=True))
        p = jnp.where(mask, jnp.exp(e - m_new), 0.0)
        scale = jnp.exp(m_old - m_new)
        s_ref[pl.ds(w, wsz), :] = (s_ref[pl.ds(w, wsz), :] * scale
                                   + jnp.sum(p, axis=1, keepdims=True))
        ph, plo = _split(p)

        def dp(u, v):
            return jax.lax.dot_general(u, v, (((1,), (0,)), ((), ())),
                                       preferred_element_type=jnp.float32)

        u_ref[pl.ds(w, wsz), :] = (u_ref[pl.ds(w, wsz), :] * scale
                                   + dp(ph, xh) + dp(ph, xl) + dp(plo, xh))
        m_ref[pl.ds(w, wsz), :] = m_new

    @pl.when(fast)
    def _fast():
        update(w8 * 8, W)

    @pl.when(jnp.logical_not(fast))
    def _general():
        update(0, B)

    @pl.when(i == NB - 1)
    def _finalize():
        r = u_ref[...] / (s_ref[...] + 1e-16)
        r_ref[...] = r

        @pl.when(t == T - 1)
        def _write_out():
            out_ref[:, :D] = h_ref[...]
            out_ref[:, D:] = r


def kernel(x, batch, W_ih, W_hh, b_ih, b_hh):
    batch = batch.astype(jnp.int32)
    batch3 = batch.reshape(NB, 1, BLK)
    idx = jnp.arange(NB)
    lo = batch[idx * BLK]
    hi = batch[idx * BLK + (BLK - 1)]
    w8 = lo // 8
    fast = (hi < w8 * 8 + W).astype(jnp.int32)
    win = jnp.stack([w8, fast], axis=1).astype(jnp.int32)
    wq = W_ih.T[:D] + W_hh.T
    wr = W_ih.T[D:]
    bias = (b_ih + b_hh).reshape(1, 4 * D)
    grid_spec = pltpu.PrefetchScalarGridSpec(
        num_scalar_prefetch=1,
        grid=(T, NB),
        in_specs=[
            pl.BlockSpec((1, 1, BLK), lambda t, i, w: (i, 0, 0)),
            pl.BlockSpec((BLK, D), lambda t, i, w: (i, 0)),
            pl.BlockSpec((D, 4 * D), lambda t, i, w: (0, 0)),
            pl.BlockSpec((D, 4 * D), lambda t, i, w: (0, 0)),
            pl.BlockSpec((1, 4 * D), lambda t, i, w: (0, 0)),
        ],
        out_specs=pl.BlockSpec((B, 2 * D), lambda t, i, w: (0, 0)),
        scratch_shapes=[
            pltpu.VMEM((B, D), jnp.float32),
            pltpu.VMEM((B, D), jnp.float32),
            pltpu.VMEM((B, D), jnp.float32),
            pltpu.VMEM((B, 1), jnp.float32),
            pltpu.VMEM((B, 1), jnp.float32),
            pltpu.VMEM((B, D), jnp.float32),
        ],
    )
    return pl.pallas_call(
        _body,
        grid_spec=grid_spec,
        out_shape=jax.ShapeDtypeStruct((B, 2 * D), jnp.float32),
        compiler_params=pltpu.CompilerParams(
            dimension_semantics=("arbitrary", "arbitrary"),
        ),
    )(win, batch3, x, wq, wr, bias)
